# Initial kernel scaffold; baseline (speedup 1.0000x reference)
import jax
import jax.numpy as jnp
from jax.experimental import pallas as pl


def kernel(fmap1, fmap2, xyz1, xyz2, truncate_k):
    raise NotImplementedError("write your pallas kernel here")


# speedup vs baseline: 1.0000x; 1.0000x over previous
import functools

import jax
import jax.numpy as jnp
from jax.experimental import pallas as pl
from jax.experimental.pallas import tpu as pltpu


def _corr_kernel(f1_ref, f2_ref, out_ref, *, scale):
    out_ref[...] = jax.lax.dot_general(
        f1_ref[...], f2_ref[...],
        dimension_numbers=(((1,), (0,)), ((), ())),
        preferred_element_type=jnp.float32,
    ) * scale


def _corr(f1t, f2, blk=512):
    N1, C = f1t.shape
    N2 = f2.shape[1]
    scale = 1.0 / jnp.sqrt(jnp.float32(C)).astype(jnp.float32)
    grid = (N1 // blk,)
    return pl.pallas_call(
        functools.partial(_corr_kernel, scale=float(1.0 / (C ** 0.5))),
        grid=grid,
        in_specs=[
            pl.BlockSpec((blk, C), lambda i: (i, 0)),
            pl.BlockSpec((C, N2), lambda i: (0, 0)),
        ],
        out_specs=pl.BlockSpec((blk, N2), lambda i: (i, 0)),
        out_shape=jax.ShapeDtypeStruct((N1, N2), jnp.float32),
    )(f1t, f2)


def kernel(fmap1, fmap2, xyz1, xyz2, truncate_k):
    B, C, N1 = fmap1.shape
    N2 = fmap2.shape[2]
    K = N2 // 4
    f1t = fmap1[0].T
    f2 = fmap2[0]
    corr = _corr(f1t, f2)[None]
    vals, idx = jax.lax.top_k(corr, K)
    nbr = jax.vmap(lambda x, i: x[i])(xyz2, idx)
    weights = jax.nn.softmax(vals, axis=-1)
    pred = jnp.einsum('bnk,bnkc->bnc', weights, nbr)
    flow = pred - xyz1
    return flow, vals, idx


# speedup vs baseline: 5.2568x; 5.2566x over previous
import functools

import jax
import jax.numpy as jnp
from jax.experimental import pallas as pl
from jax.experimental.pallas import tpu as pltpu

_LANES = 128


def _t_swap(x, dp):
    r, t, w = x.shape
    a = x.reshape(r, t // (2 * dp), 2, dp, w)
    b = jnp.concatenate([a[:, :, 1:2], a[:, :, 0:1]], axis=2)
    return b.reshape(r, t, w)


def _cmpx(v, ix, pv, pix, tm):
    nv = jnp.where(tm, jnp.maximum(v, pv), jnp.minimum(v, pv))
    chosen = (tm & (v >= pv)) | ((~tm) & (v <= pv))
    nix = jnp.where(chosen, ix, pix)
    return nv, nix


def _lane_partner(x, d, wl):
    up = pltpu.roll(x, d, 2)
    dn = pltpu.roll(x, wl - d, 2)
    return up, dn


def _lane_stage(v, ix, s, dirbit, l_iota):
    wl = v.shape[2]
    d = jnp.int32(1) << s
    lowbit = (l_iota >> s) & 1
    is_low = lowbit == 0
    upv, dnv = _lane_partner(v, d, wl)
    pv = jnp.where(is_low, dnv, upv)
    upi, dni = _lane_partner(ix, d, wl)
    pix = jnp.where(is_low, dni, upi)
    tm = (lowbit ^ dirbit) == 0
    return _cmpx(v, ix, pv, pix, tm)


def _t_stage(v, ix, sp, L, t_iota, e):
    lowbit = (t_iota >> sp) & 1
    dirbit = (e >> L) & 1
    pv = _t_swap(v, 1 << sp)
    pix = _t_swap(ix, 1 << sp)
    tm = (lowbit ^ dirbit) == 0
    return _cmpx(v, ix, pv, pix, tm)


def _merge_tail(v, ix, L, t_iota, l_iota, e):
    t = v.shape[1]
    max_dp = min(t // 2, (1 << L) // (2 * _LANES))
    sp = max_dp.bit_length() - 1 if max_dp > 0 else -1
    while sp >= 0:
        v, ix = _t_stage(v, ix, sp, L, t_iota, e)
        sp -= 1

    dirbit = (e >> L) & 1

    def body(i, c):
        vv, ii = c
        return _lane_stage(vv, ii, jnp.int32(6 - i), dirbit, l_iota)

    return jax.lax.fori_loop(0, 7, body, (v, ix))


def _combine(v, ix, tc):
    r, t, w = v.shape
    a = v.reshape(r, t // (2 * tc), 2, tc, w)
    ai = ix.reshape(r, t // (2 * tc), 2, tc, w)
    x, y = a[:, :, 0], a[:, :, 1]
    xi, yi = ai[:, :, 0], ai[:, :, 1]
    nv = jnp.maximum(x, y)
    nix = jnp.where(x >= y, xi, yi)
    return nv.reshape(r, t // 2, w), nix.reshape(r, t // 2, w)


def _iotas(t):
    l_iota = jax.lax.broadcasted_iota(jnp.int32, (1, t, _LANES), 2)
    t_iota = jax.lax.broadcasted_iota(jnp.int32, (1, t, _LANES), 1)
    e = t_iota * _LANES + l_iota
    return l_iota, t_iota, e


def _rsf_block_kernel(f1t_ref, f2_ref, xyz2_ref, xyz1_ref,
                      vals_ref, idx_ref, flow_ref, *, n2):
    T = n2 // _LANES
    KL = (n2 // 4).bit_length() - 1
    scale = 1.0 / (f2_ref.shape[0] ** 0.5)

    corr = jax.lax.dot_general(
        f1t_ref[...], f2_ref[...],
        dimension_numbers=(((1,), (0,)), ((), ())),
        preferred_element_type=jnp.float32,
    ) * scale
    R = corr.shape[0]

    l_iota, t_iota, e = _iotas(T)
    v = corr.reshape(R, T, _LANES)
    ix = jnp.broadcast_to(e, (R, T, _LANES))

    n_lane_build = sum(range(1, min(7, KL) + 1))

    def build_body(_, c):
        vv, ii, L, s = c
        dirbit = (e >> L) & 1
        vv, ii = _lane_stage(vv, ii, s, dirbit, l_iota)
        nL = jnp.where(s == 0, L + 1, L)
        ns = jnp.where(s == 0, L, s - 1)
        return vv, ii, nL, ns

    v, ix, _, _ = jax.lax.fori_loop(
        0, n_lane_build, build_body,
        (v, ix, jnp.int32(1), jnp.int32(0)))

    for L in range(8, KL + 1):
        for sp in range(L - 8, -1, -1):
            v, ix = _t_stage(v, ix, sp, L, t_iota, e)
        dirbit = (e >> L) & 1

        def lane_body(i, c, _dirbit=dirbit):
            vv, ii = c
            return _lane_stage(vv, ii, jnp.int32(6 - i), _dirbit, l_iota)

        v, ix = jax.lax.fori_loop(0, 7, lane_body, (v, ix))

    tc = (n2 // 4) // _LANES
    while v.shape[1] > tc:
        v, ix = _combine(v, ix, tc)
        t_now = v.shape[1]
        l_i, t_i, e_i = _iotas(t_now)
        v, ix = _merge_tail(v, ix, KL, t_i, l_i, e_i)

    vals_ref[...] = v.reshape(R, n2 // 4)
    idx_ref[...] = ix.reshape(R, n2 // 4)

    thr = v[:, tc - 1:tc, _LANES - 1:_LANES].reshape(R, 1)
    vmax = v[:, 0:1, 0:1].reshape(R, 1)
    w = jnp.where(corr >= thr, jnp.exp(corr - vmax), 0.0)
    denom = jnp.sum(w, axis=1, keepdims=True)
    pred = jax.lax.dot_general(
        w, xyz2_ref[...],
        dimension_numbers=(((1,), (0,)), ((), ())),
        preferred_element_type=jnp.float32,
    )
    flow_ref[...] = pred / denom - xyz1_ref[...]


def _rsf_pallas(f1t, f2, xyz1, xyz2, blk=32, interpret=False):
    n1, c = f1t.shape
    n2 = f2.shape[1]
    k = n2 // 4
    grid = (n1 // blk,)
    return pl.pallas_call(
        functools.partial(_rsf_block_kernel, n2=n2),
        grid=grid,
        in_specs=[
            pl.BlockSpec((blk, c), lambda i: (i, 0)),
            pl.BlockSpec((c, n2), lambda i: (0, 0)),
            pl.BlockSpec((n2, 3), lambda i: (0, 0)),
            pl.BlockSpec((blk, 3), lambda i: (i, 0)),
        ],
        out_specs=[
            pl.BlockSpec((blk, k), lambda i: (i, 0)),
            pl.BlockSpec((blk, k), lambda i: (i, 0)),
            pl.BlockSpec((blk, 3), lambda i: (i, 0)),
        ],
        out_shape=[
            jax.ShapeDtypeStruct((n1, k), jnp.float32),
            jax.ShapeDtypeStruct((n1, k), jnp.int32),
            jax.ShapeDtypeStruct((n1, 3), jnp.float32),
        ],
        interpret=interpret,
    )(f1t, f2, xyz2, xyz1)


def kernel(fmap1, fmap2, xyz1, xyz2, truncate_k):
    B, C, N1 = fmap1.shape
    N2 = fmap2.shape[2]
    f1t = fmap1[0].T
    f2 = fmap2[0]
    vals, idx, flow = _rsf_pallas(f1t, f2, xyz1[0], xyz2[0])
    return flow[None], vals[None], idx[None]
